# initial kernel scaffold (unmeasured)
import jax
import jax.numpy as jnp
from jax import lax
from jax.experimental import pallas as pl
from jax.experimental.pallas import tpu as pltpu

NZ = 4
KB = 512
N_STEPS = NZ - 1


def kernel(dy, W):
    M, K = dy.shape
    D = W.shape[0]
    nk = K // KB
    half = M // 2
    chunk = half // NZ

    def body(dy_ref, w_ref, out_ref, recv_buf, send_buf, recv_sems, send_sems):
        k = pl.program_id(0)

        a = dy_ref[:, :].astype(jnp.bfloat16)
        b = w_ref[:, :].astype(jnp.bfloat16)
        prod = lax.dot_general(
            a, b, (((1,), (1,)), ((), ())),
            preferred_element_type=jnp.float32,
        )

        @pl.when(k == 0)
        def _():
            out_ref[:, :] = prod

        @pl.when(k != 0)
        def _():
            out_ref[:, :] = out_ref[:, :] + prod

        @pl.when(k == nk - 1)
        def _():
            my_x = lax.axis_index("x")
            my_y = lax.axis_index("y")
            my_z = lax.axis_index("z")

            dirs = ((0, 1, 0), (1, -1, half))

            def tgt(dd):
                return (my_x, my_y, (my_z + dd) % NZ)

            def rows(base, c):
                return pl.ds(base + c * chunk, chunk)

            barrier = pltpu.get_barrier_semaphore()
            for dd in (1, -1):
                pl.semaphore_signal(
                    barrier, inc=1,
                    device_id=tgt(dd),
                    device_id_type=pl.DeviceIdType.MESH,
                )
            pl.semaphore_wait(barrier, 2)

            started = []

            for s in range(N_STEPS):
                hop = []
                for d, dd, base in dirs:
                    send_c = (my_z + ((-dd * s) % NZ)) % NZ
                    send_buf[d, s] = out_ref[rows(base, send_c), :].astype(
                        jnp.bfloat16
                    )
                    rdma = pltpu.make_async_remote_copy(
                        src_ref=send_buf.at[d, s],
                        dst_ref=recv_buf.at[d, s],
                        send_sem=send_sems.at[d, s],
                        recv_sem=recv_sems.at[d, s],
                        device_id=tgt(dd),
                        device_id_type=pl.DeviceIdType.MESH,
                    )
                    rdma.start()
                    hop.append(rdma)
                    started.append(rdma)
                for (d, dd, base), rdma in zip(dirs, hop):
                    rdma.wait_recv()
                    recv_c = (my_z + ((-dd * (s + 1)) % NZ)) % NZ
                    out_ref[rows(base, recv_c), :] = (
                        out_ref[rows(base, recv_c), :]
                        + recv_buf[d, s].astype(jnp.float32)
                    )

            for d, dd, base in dirs:
                own_c = (my_z + (dd % NZ)) % NZ
                send_buf[d, N_STEPS] = out_ref[rows(base, own_c), :].astype(
                    jnp.bfloat16
                )
            for s in range(N_STEPS):
                hop = []
                for d, dd, base in dirs:
                    src = (
                        send_buf.at[d, N_STEPS]
                        if s == 0
                        else recv_buf.at[d, N_STEPS + s - 1]
                    )
                    rdma = pltpu.make_async_remote_copy(
                        src_ref=src,
                        dst_ref=recv_buf.at[d, N_STEPS + s],
                        send_sem=send_sems.at[d, N_STEPS + s],
                        recv_sem=recv_sems.at[d, N_STEPS + s],
                        device_id=tgt(dd),
                        device_id_type=pl.DeviceIdType.MESH,
                    )
                    rdma.start()
                    hop.append(rdma)
                    started.append(rdma)
                for (d, dd, base), rdma in zip(dirs, hop):
                    rdma.wait_recv()
                    recv_c = (my_z + ((-dd * s) % NZ)) % NZ
                    out_ref[rows(base, recv_c), :] = recv_buf[
                        d, N_STEPS + s
                    ].astype(jnp.float32)

            for rdma in started:
                rdma.wait_send()

        return

    return pl.pallas_call(
        body,
        grid=(nk,),
        in_specs=[
            pl.BlockSpec((M, KB), lambda k: (0, k)),
            pl.BlockSpec((D, KB), lambda k: (0, k)),
        ],
        out_specs=pl.BlockSpec((M, D), lambda k: (0, 0)),
        out_shape=jax.ShapeDtypeStruct((M, D), jnp.float32),
        scratch_shapes=[
            pltpu.VMEM((2, 2 * N_STEPS, chunk, D), jnp.bfloat16),
            pltpu.VMEM((2, N_STEPS + 1, chunk, D), jnp.bfloat16),
            pltpu.SemaphoreType.DMA((2, 2 * N_STEPS)),
            pltpu.SemaphoreType.DMA((2, 2 * N_STEPS)),
        ],
        compiler_params=pltpu.CompilerParams(
            dimension_semantics=("arbitrary",),
            collective_id=0,
        ),
    )(dy, W)


# baseline (device time: 288368 ns/iter reference)
import jax
import jax.numpy as jnp
from jax import lax
from jax.experimental import pallas as pl
from jax.experimental.pallas import tpu as pltpu

NZ = 4
KB = 512
N_STEPS = NZ - 1


def kernel(dy, W):
    M, K = dy.shape
    D = W.shape[0]
    nk = K // KB
    half = M // 2
    chunk = half // NZ

    def body(dy_ref, w_ref, out_ref, recv_buf, send_buf, recv_sems, send_sems):
        k = pl.program_id(0)

        b = w_ref[:, :].astype(jnp.bfloat16)
        MB = 512
        for mi in range(M // MB):
            sl = pl.ds(mi * MB, MB)
            a = dy_ref[sl, :].astype(jnp.bfloat16)
            prod = lax.dot_general(
                a, b, (((1,), (1,)), ((), ())),
                preferred_element_type=jnp.float32,
            )

            @pl.when(k == 0)
            def _():
                out_ref[sl, :] = prod

            @pl.when(k != 0)
            def _():
                out_ref[sl, :] = out_ref[sl, :] + prod

        @pl.when(k == nk - 1)
        def _():
            my_x = lax.axis_index("x")
            my_y = lax.axis_index("y")
            my_z = lax.axis_index("z")

            dirs = ((0, 1, 0), (1, -1, half))

            def tgt(dd):
                return (my_x, my_y, (my_z + dd) % NZ)

            def rows(base, c):
                return pl.ds(base + c * chunk, chunk)

            barrier = pltpu.get_barrier_semaphore()
            for dd in (1, -1):
                pl.semaphore_signal(
                    barrier, inc=1,
                    device_id=tgt(dd),
                    device_id_type=pl.DeviceIdType.MESH,
                )
            pl.semaphore_wait(barrier, 2)

            started = []

            for s in range(N_STEPS):
                hop = []
                for d, dd, base in dirs:
                    send_c = (my_z + ((-dd * s) % NZ)) % NZ
                    send_buf[d, s] = out_ref[rows(base, send_c), :].astype(
                        jnp.bfloat16
                    )
                    rdma = pltpu.make_async_remote_copy(
                        src_ref=send_buf.at[d, s],
                        dst_ref=recv_buf.at[d, s],
                        send_sem=send_sems.at[d, s],
                        recv_sem=recv_sems.at[d, s],
                        device_id=tgt(dd),
                        device_id_type=pl.DeviceIdType.MESH,
                    )
                    rdma.start()
                    hop.append(rdma)
                    started.append(rdma)
                for (d, dd, base), rdma in zip(dirs, hop):
                    rdma.wait_recv()
                    recv_c = (my_z + ((-dd * (s + 1)) % NZ)) % NZ
                    out_ref[rows(base, recv_c), :] = (
                        out_ref[rows(base, recv_c), :]
                        + recv_buf[d, s].astype(jnp.float32)
                    )

            for d, dd, base in dirs:
                own_c = (my_z + (dd % NZ)) % NZ
                send_buf[d, N_STEPS] = out_ref[rows(base, own_c), :].astype(
                    jnp.bfloat16
                )
            for s in range(N_STEPS):
                hop = []
                for d, dd, base in dirs:
                    src = (
                        send_buf.at[d, N_STEPS]
                        if s == 0
                        else recv_buf.at[d, N_STEPS + s - 1]
                    )
                    rdma = pltpu.make_async_remote_copy(
                        src_ref=src,
                        dst_ref=recv_buf.at[d, N_STEPS + s],
                        send_sem=send_sems.at[d, N_STEPS + s],
                        recv_sem=recv_sems.at[d, N_STEPS + s],
                        device_id=tgt(dd),
                        device_id_type=pl.DeviceIdType.MESH,
                    )
                    rdma.start()
                    hop.append(rdma)
                    started.append(rdma)
                for (d, dd, base), rdma in zip(dirs, hop):
                    rdma.wait_recv()
                    recv_c = (my_z + ((-dd * s) % NZ)) % NZ
                    out_ref[rows(base, recv_c), :] = recv_buf[
                        d, N_STEPS + s
                    ].astype(jnp.float32)

            for rdma in started:
                rdma.wait_send()

        return

    return pl.pallas_call(
        body,
        grid=(nk,),
        in_specs=[
            pl.BlockSpec((M, KB), lambda k: (0, k)),
            pl.BlockSpec((D, KB), lambda k: (0, k)),
        ],
        out_specs=pl.BlockSpec((M, D), lambda k: (0, 0)),
        out_shape=jax.ShapeDtypeStruct((M, D), jnp.float32),
        scratch_shapes=[
            pltpu.VMEM((2, 2 * N_STEPS, chunk, D), jnp.bfloat16),
            pltpu.VMEM((2, N_STEPS + 1, chunk, D), jnp.bfloat16),
            pltpu.SemaphoreType.DMA((2, 2 * N_STEPS)),
            pltpu.SemaphoreType.DMA((2, 2 * N_STEPS)),
        ],
        compiler_params=pltpu.CompilerParams(
            dimension_semantics=("arbitrary",),
            collective_id=0,
            vmem_limit_bytes=64 * 1024 * 1024,
        ),
    )(dy, W)


# device time: 216423 ns/iter; 1.3324x vs baseline; 1.3324x over previous
import jax
import jax.numpy as jnp
from jax import lax
from jax.experimental import pallas as pl
from jax.experimental.pallas import tpu as pltpu

NZ = 4
NQ = 4
KB = 512
SUB = 128


def kernel(dy, W):
    M, K = dy.shape
    D = W.shape[0]
    nk = K // KB

    def body(
        dy_ref, w_ref, out_ref,
        zrs_send, zrs_recv, zag_recv, own_send, xy_recv,
        zrs_send_sems, zrs_recv_sems,
        zag_send_sems, zag_recv_sems,
        xyo_send_sems, xyf_send_sems, xy_recv_sems,
    ):
        k = pl.program_id(0)

        b = w_ref[:, :].astype(jnp.bfloat16)
        MB = 512
        for mi in range(M // MB):
            sl = pl.ds(mi * MB, MB)
            a = dy_ref[sl, :].astype(jnp.bfloat16)
            prod = lax.dot_general(
                a, b, (((1,), (1,)), ((), ())),
                preferred_element_type=jnp.float32,
            )

            @pl.when(k == 0)
            def _():
                out_ref[sl, :] = prod

            @pl.when(k != 0)
            def _():
                out_ref[sl, :] = out_ref[sl, :] + prod

        @pl.when(k == nk - 1)
        def _():
            my_x = lax.axis_index("x")
            my_y = lax.axis_index("y")
            my_z = lax.axis_index("z")
            q = my_x * 2 + my_y

            def rows(qq, j):
                return pl.ds(qq * (NZ * SUB) + j * SUB, SUB)

            def z_peer(j):
                return (my_x, my_y, j)

            def xy_peer(p):
                return (p // 2, p % 2, my_z)

            barrier = pltpu.get_barrier_semaphore()
            for j in range(NZ):
                @pl.when(j != my_z)
                def _(j=j):
                    pl.semaphore_signal(
                        barrier, inc=1, device_id=z_peer(j),
                        device_id_type=pl.DeviceIdType.MESH,
                    )
            for p in range(NQ):
                @pl.when(p != q)
                def _(p=p):
                    pl.semaphore_signal(
                        barrier, inc=1, device_id=xy_peer(p),
                        device_id_type=pl.DeviceIdType.MESH,
                    )
            pl.semaphore_wait(barrier, NZ - 1 + NQ - 1)

            for j in range(NZ):
                @pl.when(j != my_z)
                def _(j=j):
                    zrs_send[j, :, :] = out_ref[rows(q, j), :].astype(jnp.bfloat16)
                    pltpu.make_async_remote_copy(
                        src_ref=zrs_send.at[j],
                        dst_ref=zrs_recv.at[my_z],
                        send_sem=zrs_send_sems.at[j],
                        recv_sem=zrs_recv_sems.at[my_z],
                        device_id=z_peer(j),
                        device_id_type=pl.DeviceIdType.MESH,
                    ).start()
            for j in range(NZ):
                @pl.when(j != my_z)
                def _(j=j):
                    pltpu.make_async_remote_copy(
                        src_ref=zrs_send.at[j],
                        dst_ref=zrs_recv.at[j],
                        send_sem=zrs_send_sems.at[j],
                        recv_sem=zrs_recv_sems.at[j],
                        device_id=z_peer(j),
                        device_id_type=pl.DeviceIdType.MESH,
                    ).wait_recv()
                    out_ref[rows(q, my_z), :] = (
                        out_ref[rows(q, my_z), :]
                        + zrs_recv[j, :, :].astype(jnp.float32)
                    )

            own_send[:, :] = out_ref[rows(q, my_z), :].astype(jnp.bfloat16)
            for j in range(NZ):
                @pl.when(j != my_z)
                def _(j=j):
                    pltpu.make_async_remote_copy(
                        src_ref=own_send,
                        dst_ref=zag_recv.at[my_z],
                        send_sem=zag_send_sems.at[j],
                        recv_sem=zag_recv_sems.at[my_z],
                        device_id=z_peer(j),
                        device_id_type=pl.DeviceIdType.MESH,
                    ).start()
            for p in range(NQ):
                @pl.when(p != q)
                def _(p=p):
                    pltpu.make_async_remote_copy(
                        src_ref=own_send,
                        dst_ref=xy_recv.at[q, my_z],
                        send_sem=xyo_send_sems.at[p],
                        recv_sem=xy_recv_sems.at[q, my_z],
                        device_id=xy_peer(p),
                        device_id_type=pl.DeviceIdType.MESH,
                    ).start()

            for j in range(NZ):
                @pl.when(j != my_z)
                def _(j=j):
                    pltpu.make_async_remote_copy(
                        src_ref=own_send,
                        dst_ref=zag_recv.at[j],
                        send_sem=zag_send_sems.at[j],
                        recv_sem=zag_recv_sems.at[j],
                        device_id=z_peer(j),
                        device_id_type=pl.DeviceIdType.MESH,
                    ).wait_recv()
                    for p in range(NQ):
                        @pl.when(p != q)
                        def _(p=p):
                            pltpu.make_async_remote_copy(
                                src_ref=zag_recv.at[j],
                                dst_ref=xy_recv.at[q, j],
                                send_sem=xyf_send_sems.at[p, j],
                                recv_sem=xy_recv_sems.at[q, j],
                                device_id=xy_peer(p),
                                device_id_type=pl.DeviceIdType.MESH,
                            ).start()
                    out_ref[rows(q, j), :] = zag_recv[j, :, :].astype(jnp.float32)

            for p in range(NQ):
                @pl.when(p != q)
                def _(p=p):
                    for j in range(NZ):
                        pltpu.make_async_remote_copy(
                            src_ref=own_send,
                            dst_ref=xy_recv.at[p, j],
                            send_sem=xyo_send_sems.at[p],
                            recv_sem=xy_recv_sems.at[p, j],
                            device_id=xy_peer(p),
                            device_id_type=pl.DeviceIdType.MESH,
                        ).wait_recv()
                        out_ref[rows(p, j), :] = xy_recv[p, j, :, :].astype(
                            jnp.float32
                        )

            for j in range(NZ):
                @pl.when(j != my_z)
                def _(j=j):
                    pltpu.make_async_remote_copy(
                        src_ref=zrs_send.at[j], dst_ref=zrs_recv.at[j],
                        send_sem=zrs_send_sems.at[j],
                        recv_sem=zrs_recv_sems.at[j],
                        device_id=z_peer(j),
                        device_id_type=pl.DeviceIdType.MESH,
                    ).wait_send()
                    pltpu.make_async_remote_copy(
                        src_ref=own_send, dst_ref=zag_recv.at[j],
                        send_sem=zag_send_sems.at[j],
                        recv_sem=zag_recv_sems.at[j],
                        device_id=z_peer(j),
                        device_id_type=pl.DeviceIdType.MESH,
                    ).wait_send()
            for p in range(NQ):
                @pl.when(p != q)
                def _(p=p):
                    pltpu.make_async_remote_copy(
                        src_ref=own_send, dst_ref=xy_recv.at[p, 0],
                        send_sem=xyo_send_sems.at[p],
                        recv_sem=xy_recv_sems.at[p, 0],
                        device_id=xy_peer(p),
                        device_id_type=pl.DeviceIdType.MESH,
                    ).wait_send()
                    for j in range(NZ):
                        @pl.when(j != my_z)
                        def _(p=p, j=j):
                            pltpu.make_async_remote_copy(
                                src_ref=zag_recv.at[j],
                                dst_ref=xy_recv.at[p, j],
                                send_sem=xyf_send_sems.at[p, j],
                                recv_sem=xy_recv_sems.at[p, j],
                                device_id=xy_peer(p),
                                device_id_type=pl.DeviceIdType.MESH,
                            ).wait_send()

        return

    return pl.pallas_call(
        body,
        grid=(nk,),
        in_specs=[
            pl.BlockSpec((M, KB), lambda k: (0, k)),
            pl.BlockSpec((D, KB), lambda k: (0, k)),
        ],
        out_specs=pl.BlockSpec((M, D), lambda k: (0, 0)),
        out_shape=jax.ShapeDtypeStruct((M, D), jnp.float32),
        scratch_shapes=[
            pltpu.VMEM((NZ, SUB, D), jnp.bfloat16),
            pltpu.VMEM((NZ, SUB, D), jnp.bfloat16),
            pltpu.VMEM((NZ, SUB, D), jnp.bfloat16),
            pltpu.VMEM((SUB, D), jnp.bfloat16),
            pltpu.VMEM((NQ, NZ, SUB, D), jnp.bfloat16),
            pltpu.SemaphoreType.DMA((NZ,)),
            pltpu.SemaphoreType.DMA((NZ,)),
            pltpu.SemaphoreType.DMA((NZ,)),
            pltpu.SemaphoreType.DMA((NZ,)),
            pltpu.SemaphoreType.DMA((NQ,)),
            pltpu.SemaphoreType.DMA((NQ, NZ)),
            pltpu.SemaphoreType.DMA((NQ, NZ)),
        ],
        compiler_params=pltpu.CompilerParams(
            dimension_semantics=("arbitrary",),
            collective_id=0,
            vmem_limit_bytes=64 * 1024 * 1024,
        ),
    )(dy, W)


# device time: 120066 ns/iter; 2.4017x vs baseline; 1.8025x over previous
import jax
import jax.numpy as jnp
from jax import lax
from jax.experimental import pallas as pl
from jax.experimental.pallas import tpu as pltpu

NZ = 4
NQ = 4
KB = 1024
PR = 256
PC = 1024


def kernel(dy, W):
    M, K = dy.shape
    D = W.shape[0]
    nk = K // KB

    def body(
        q_ref, dy_ref, w_ref, out_ref,
        zrs_send, zrs_recv, zag_recv, own_send, xy_recv,
        zrs_send_sems, zrs_recv_sems,
        zag_dn_sems, zag_up_sems, zag_recv_sems,
        xds_sems, yds_sems, xfw_sems, yfw_sems, xy_recv_sems,
    ):
        k = pl.program_id(0)
        H = NZ * PR
        rsl = pl.ds(q_ref[0] * H, H)
        csl = pl.ds(q_ref[1] * H, H)

        b = w_ref[:, :].astype(jnp.bfloat16)
        a = dy_ref[:, :].astype(jnp.bfloat16)
        prod = lax.dot_general(
            a, b, (((1,), (1,)), ((), ())),
            preferred_element_type=jnp.float32,
        )

        @pl.when(k == 0)
        def _():
            out_ref[rsl, csl] = prod

        @pl.when(k != 0)
        def _():
            out_ref[rsl, csl] = out_ref[rsl, csl] + prod

        @pl.when(k == nk - 1)
        def _():
            my_x = lax.axis_index("x")
            my_y = lax.axis_index("y")
            my_z = lax.axis_index("z")
            q = my_x * 2 + my_y

            def piece(pr, pc, j):
                return (
                    pl.ds(pr * (NZ * PR) + j * PR, PR),
                    pl.ds(pc * (NZ * PR), NZ * PR),
                )

            def z_peer(j):
                return (my_x, my_y, j)

            x_nbr = (1 - my_x, my_y, my_z)
            y_nbr = (my_x, 1 - my_y, my_z)
            xcol = (1 - my_x) * 2 + my_y
            ycol = my_x * 2 + (1 - my_y)
            dcol = (1 - my_x) * 2 + (1 - my_y)
            A = (0, 2)
            B = (1, 3)

            barrier = pltpu.get_barrier_semaphore()
            for j in range(NZ):
                @pl.when(j != my_z)
                def _(j=j):
                    pl.semaphore_signal(
                        barrier, inc=1, device_id=z_peer(j),
                        device_id_type=pl.DeviceIdType.MESH,
                    )
            for nbr in (x_nbr, y_nbr):
                pl.semaphore_signal(
                    barrier, inc=1, device_id=nbr,
                    device_id_type=pl.DeviceIdType.MESH,
                )
            pl.semaphore_wait(barrier, NZ - 1 + 2)

            for j in range(NZ):
                @pl.when(j != my_z)
                def _(j=j):
                    zrs_send[j, :, :] = out_ref[piece(my_x, my_y, j)].astype(jnp.bfloat16)
                    pltpu.make_async_remote_copy(
                        src_ref=zrs_send.at[j],
                        dst_ref=zrs_recv.at[my_z],
                        send_sem=zrs_send_sems.at[j],
                        recv_sem=zrs_recv_sems.at[my_z],
                        device_id=z_peer(j),
                        device_id_type=pl.DeviceIdType.MESH,
                    ).start()
            for j in range(NZ):
                @pl.when(j != my_z)
                def _(j=j):
                    pltpu.make_async_remote_copy(
                        src_ref=zrs_send.at[j],
                        dst_ref=zrs_recv.at[j],
                        send_sem=zrs_send_sems.at[j],
                        recv_sem=zrs_recv_sems.at[j],
                        device_id=z_peer(j),
                        device_id_type=pl.DeviceIdType.MESH,
                    ).wait_recv()
                    out_ref[piece(my_x, my_y, my_z)] = (
                        out_ref[piece(my_x, my_y, my_z)]
                        + zrs_recv[j, :, :].astype(jnp.float32)
                    )

            own_send[:, :] = out_ref[piece(my_x, my_y, my_z)].astype(jnp.bfloat16)

            def zag_hop(src, piece_j, to_z, sems):
                return pltpu.make_async_remote_copy(
                    src_ref=src,
                    dst_ref=zag_recv.at[piece_j],
                    send_sem=sems.at[piece_j],
                    recv_sem=zag_recv_sems.at[piece_j],
                    device_id=(my_x, my_y, to_z),
                    device_id_type=pl.DeviceIdType.MESH,
                )

            @pl.when(my_z > 0)
            def _():
                zag_hop(own_send, my_z, my_z - 1, zag_dn_sems).start()

            @pl.when(my_z < NZ - 1)
            def _():
                zag_hop(own_send, my_z, my_z + 1, zag_up_sems).start()
            def plane_send(src, col, j, nbr, sem):
                return pltpu.make_async_remote_copy(
                    src_ref=src,
                    dst_ref=xy_recv.at[col, j],
                    send_sem=sem.at[j],
                    recv_sem=xy_recv_sems.at[col, j],
                    device_id=nbr,
                    device_id_type=pl.DeviceIdType.MESH,
                )

            plane_send(own_send, q, my_z, x_nbr, xds_sems).start()
            plane_send(own_send, q, my_z, y_nbr, yds_sems).start()

            for d in (1, 2, 3):
                for sgn in (-1, 1):
                    jt = my_z + sgn * d
                    @pl.when((jt >= 0) & (jt <= NZ - 1))
                    def _(jt=jt, sgn=sgn):
                        zag_hop(own_send, jt, my_z, zag_dn_sems).wait_recv()
                        if sgn < 0:
                            @pl.when(my_z < NZ - 1)
                            def _(jt=jt):
                                zag_hop(
                                    zag_recv.at[jt], jt, my_z + 1, zag_up_sems
                                ).start()
                        else:
                            @pl.when(my_z > 0)
                            def _(jt=jt):
                                zag_hop(
                                    zag_recv.at[jt], jt, my_z - 1, zag_dn_sems
                                ).start()
                        plane_send(zag_recv.at[jt], q, jt, x_nbr, xds_sems).start()
                        plane_send(zag_recv.at[jt], q, jt, y_nbr, yds_sems).start()
                        out_ref[piece(my_x, my_y, jt)] = zag_recv[
                            jt, :, :
                        ].astype(jnp.float32)

            for j in A + B:
                plane_send(own_send, xcol, j, x_nbr, xds_sems).wait_recv()
                if j in A:
                    plane_send(
                        xy_recv.at[xcol, j], xcol, j, y_nbr, yfw_sems
                    ).start()
                out_ref[piece(1 - my_x, my_y, j)] = xy_recv[
                    xcol, j, :, :
                ].astype(jnp.float32)
            for j in B + A:
                plane_send(own_send, ycol, j, y_nbr, yds_sems).wait_recv()
                if j in B:
                    plane_send(
                        xy_recv.at[ycol, j], ycol, j, x_nbr, xfw_sems
                    ).start()
                out_ref[piece(my_x, 1 - my_y, j)] = xy_recv[
                    ycol, j, :, :
                ].astype(jnp.float32)
            for j in range(NZ):
                plane_send(own_send, dcol, j, x_nbr, xds_sems).wait_recv()
                out_ref[piece(1 - my_x, 1 - my_y, j)] = xy_recv[
                    dcol, j, :, :
                ].astype(jnp.float32)

            for j in range(NZ):
                @pl.when(j != my_z)
                def _(j=j):
                    pltpu.make_async_remote_copy(
                        src_ref=zrs_send.at[j], dst_ref=zrs_recv.at[j],
                        send_sem=zrs_send_sems.at[j],
                        recv_sem=zrs_recv_sems.at[j],
                        device_id=z_peer(j),
                        device_id_type=pl.DeviceIdType.MESH,
                    ).wait_send()
            @pl.when(my_z > 0)
            def _():
                zag_hop(own_send, my_z, my_z - 1, zag_dn_sems).wait_send()

            @pl.when(my_z < NZ - 1)
            def _():
                zag_hop(own_send, my_z, my_z + 1, zag_up_sems).wait_send()
            for d in (1, 2, 3):
                for sgn in (-1, 1):
                    jt = my_z + sgn * d
                    if sgn < 0:
                        @pl.when((jt >= 0) & (my_z < NZ - 1))
                        def _(jt=jt):
                            zag_hop(
                                zag_recv.at[jt], jt, my_z + 1, zag_up_sems
                            ).wait_send()
                    else:
                        @pl.when((jt <= NZ - 1) & (my_z > 0))
                        def _(jt=jt):
                            zag_hop(
                                zag_recv.at[jt], jt, my_z - 1, zag_dn_sems
                            ).wait_send()
            for j in range(NZ):
                plane_send(own_send, q, j, x_nbr, xds_sems).wait_send()
                plane_send(own_send, q, j, y_nbr, yds_sems).wait_send()
                if j in A:
                    plane_send(
                        xy_recv.at[xcol, j], xcol, j, y_nbr, yfw_sems
                    ).wait_send()
                if j in B:
                    plane_send(
                        xy_recv.at[ycol, j], ycol, j, x_nbr, xfw_sems
                    ).wait_send()

        return

    grid_spec = pltpu.PrefetchScalarGridSpec(
        num_scalar_prefetch=1,
        grid=(nk,),
        in_specs=[
            pl.BlockSpec((NZ * PR, KB), lambda k, s: (s[0], k)),
            pl.BlockSpec((NZ * PR, KB), lambda k, s: (s[1], k)),
        ],
        out_specs=pl.BlockSpec((M, D), lambda k, q: (0, 0)),
        scratch_shapes=[
            pltpu.VMEM((NZ, PR, PC), jnp.bfloat16),
            pltpu.VMEM((NZ, PR, PC), jnp.bfloat16),
            pltpu.VMEM((NZ, PR, PC), jnp.bfloat16),
            pltpu.VMEM((PR, PC), jnp.bfloat16),
            pltpu.VMEM((NQ, NZ, PR, PC), jnp.bfloat16),
            pltpu.SemaphoreType.DMA((NZ,)),
            pltpu.SemaphoreType.DMA((NZ,)),
            pltpu.SemaphoreType.DMA((NZ,)),
            pltpu.SemaphoreType.DMA((NZ,)),
            pltpu.SemaphoreType.DMA((NZ,)),
            pltpu.SemaphoreType.DMA((NZ,)),
            pltpu.SemaphoreType.DMA((NZ,)),
            pltpu.SemaphoreType.DMA((NZ,)),
            pltpu.SemaphoreType.DMA((NZ,)),
            pltpu.SemaphoreType.DMA((NQ, NZ)),
        ],
    )
    s_pref = jnp.stack(
        [lax.axis_index("x"), lax.axis_index("y")]
    ).astype(jnp.int32)
    return pl.pallas_call(
        body,
        grid_spec=grid_spec,
        out_shape=jax.ShapeDtypeStruct((M, D), jnp.float32),
        compiler_params=pltpu.CompilerParams(
            dimension_semantics=("arbitrary",),
            collective_id=11,
            vmem_limit_bytes=64 * 1024 * 1024,
        ),
    )(s_pref, dy, W)
